# Initial kernel scaffold; baseline (speedup 1.0000x reference)
import jax
import jax.numpy as jnp
from jax.experimental import pallas as pl


def kernel(x, local_edge_index, global_edge_index, W1, W2_local, W2_global, gamma, beta):
    raise NotImplementedError("write your pallas kernel here")


# speedup vs baseline: 8.5449x; 8.5449x over previous
import functools

import jax
import jax.numpy as jnp
from jax import lax
from jax.experimental import pallas as pl
from jax.experimental.pallas import tpu as pltpu
from jax.experimental.pallas import tpu_sc as plsc

N = 10000
D = 128
H = D // 2
E_L = 320000
E_G = 160000
E = E_L + E_G
NC, NS = 2, 16
EPT = E // NS
K = 120
CH = EPT // K
W = 50
NWIN = CH // W
RPT = 1248
RPT_LAST = 2 * N - (NS - 1) * RPT
BN_EPS = 1e-5


def _sc_aggregate(x2, cols, rows, zrows):
    mesh = plsc.VectorSubcoreMesh(core_axis_name="c", subcore_axis_name="s",
                                  num_cores=NC, num_subcores=NS)

    @functools.partial(
        pl.kernel,
        out_type=jax.ShapeDtypeStruct((NC, 2 * N, H), jnp.float32),
        mesh=mesh,
        scratch_types=[
            pltpu.VMEM((W, K), jnp.int32),
            pltpu.VMEM((W, K), jnp.int32),
            pltpu.VMEM((K, H), jnp.float32),
            pltpu.VMEM((K, H), jnp.float32),
            pltpu.VMEM_SHARED((2 * N, H), jnp.float32),
            pltpu.SemaphoreType.DMA,
            pltpu.SemaphoreType.DMA,
        ],
        compiler_params=pltpu.CompilerParams(use_tc_tiling_on_sc=False),
    )
    def k(x2_hbm, cols_hbm, rows_hbm, z_hbm, out_hbm,
          cw, rw, g0, g1, acc, sem0, sem1):
        cid = lax.axis_index("c")
        sid = lax.axis_index("s")
        @pl.when(sid < NS - 1)
        def _():
            pltpu.sync_copy(z_hbm.at[pl.ds(0, RPT)], acc.at[pl.ds(sid * RPT, RPT)])

        @pl.when(sid == NS - 1)
        def _():
            pltpu.sync_copy(z_hbm, acc.at[pl.ds((NS - 1) * RPT, RPT_LAST)])

        plsc.subcore_barrier()

        for w in range(NWIN):
            pltpu.sync_copy(cols_hbm.at[cid, sid, pl.ds(w * W, W)], cw)
            pltpu.sync_copy(rows_hbm.at[sid, pl.ds(w * W, W)], rw)
            pltpu.async_copy(x2_hbm.at[cw.at[0]], g0, sem0)

            def pair(j, carry):
                pltpu.async_copy(x2_hbm.at[cw.at[2 * j + 1]], g1, sem1)
                pltpu.make_async_copy(x2_hbm.at[cw.at[2 * j]], g0, sem0).wait()
                pltpu.sync_copy(g0, acc.at[rw.at[2 * j]], add=True)

                @pl.when(j < W // 2 - 1)
                def _():
                    pltpu.async_copy(x2_hbm.at[cw.at[2 * j + 2]], g0, sem0)

                pltpu.make_async_copy(x2_hbm.at[cw.at[2 * j + 1]], g1, sem1).wait()
                pltpu.sync_copy(g1, acc.at[rw.at[2 * j + 1]], add=True)
                return carry

            lax.fori_loop(0, W // 2, pair, 0)
        plsc.subcore_barrier()

        @pl.when(sid < NS - 1)
        def _():
            pltpu.sync_copy(acc.at[pl.ds(sid * RPT, RPT)],
                            out_hbm.at[cid, pl.ds(sid * RPT, RPT)])

        @pl.when(sid == NS - 1)
        def _():
            pltpu.sync_copy(acc.at[pl.ds((NS - 1) * RPT, RPT_LAST)],
                            out_hbm.at[cid, pl.ds((NS - 1) * RPT, RPT_LAST)])

    return k(x2, cols, rows, zrows)


def _tc_finish_body(x_ref, parts_ref, w1t_ref, w2lt_ref, w2gt_ref,
                    gamma_ref, beta_ref, out_ref):
    f32 = jnp.float32
    out = jnp.dot(x_ref[...], w1t_ref[...], preferred_element_type=f32)
    out += jnp.dot(parts_ref[0, :N, :], w2lt_ref[:H, :], preferred_element_type=f32)
    out += jnp.dot(parts_ref[1, :N, :], w2lt_ref[H:, :], preferred_element_type=f32)
    out += jnp.dot(parts_ref[0, N:, :], w2gt_ref[:H, :], preferred_element_type=f32)
    out += jnp.dot(parts_ref[1, N:, :], w2gt_ref[H:, :], preferred_element_type=f32)
    mean = jnp.mean(out, axis=0, keepdims=True)
    var = jnp.mean(out * out, axis=0, keepdims=True) - mean * mean
    out = (out - mean) * lax.rsqrt(var + BN_EPS) * gamma_ref[...] + beta_ref[...]
    out_ref[...] = jnp.maximum(out, 0.0)


def _tc_finish(x, parts, w1t, w2lt, w2gt, gamma2d, beta2d):
    return pl.pallas_call(
        _tc_finish_body,
        out_shape=jax.ShapeDtypeStruct((N, D), jnp.float32),
    )(x, parts, w1t, w2lt, w2gt, gamma2d, beta2d)


def kernel(x, local_edge_index, global_edge_index, W1, W2_local, W2_global,
           gamma, beta):
    x2 = jnp.concatenate([x[:, :H], x[:, H:]], axis=0)
    col = jnp.concatenate([local_edge_index[1], global_edge_index[1]])
    row = jnp.concatenate([local_edge_index[0], global_edge_index[0] + N])
    cols = col[None, :] + (N * jnp.arange(NC, dtype=jnp.int32))[:, None]
    cols = cols.reshape(NC, NS, CH, K)
    rows = jnp.broadcast_to(row.reshape(1, NS, CH, K), (1, NS, CH, K))[0]
    zrows = jnp.zeros((RPT_LAST, H), dtype=jnp.float32)
    x2, cols, rows, zrows = lax.optimization_barrier((x2, cols, rows, zrows))

    parts = _sc_aggregate(x2, cols, rows, zrows)

    w1t = W1.T
    w2lt = W2_local.T
    w2gt = W2_global.T
    return _tc_finish(x, parts, w1t, w2lt, w2gt,
                      gamma.reshape(1, D), beta.reshape(1, D))
